# baseline (speedup 1.0000x reference)
import jax
import jax.numpy as jnp
from jax import lax
from jax.experimental import pallas as pl
from jax.experimental.pallas import tpu as pltpu

_B, _S, _DIN, _F, _Q = 8, 16, 256, 64, 65536
_ROWS = _B * _S
_INV_T = 1.0 / 0.07
_CB = 1024
_NQ = _Q // _CB
_GRID = _NQ + 1


def _moco_body(ptr_ref, xq_ref, xk_ref, wq_ref, wk_ref, qblk_ref,
               out_l_ref, out_q_ref, qf_ref, kf_ref, carry_ref):
    i = pl.program_id(0)

    @pl.when(i == 0)
    def _():
        qf = lax.dot(xq_ref[...], wq_ref[...])
        qf = qf / jnp.sqrt(jnp.sum(qf * qf, axis=-1, keepdims=True) + 1e-12)
        kf = lax.dot(xk_ref[...], wk_ref[...])
        kf = kf / jnp.sqrt(jnp.sum(kf * kf, axis=-1, keepdims=True) + 1e-12)
        qf_ref[...] = qf
        kf_ref[...] = kf
        carry_ref[...] = jnp.sum(qf * kf, axis=-1, keepdims=True)

    qblk = qblk_ref[...]
    neg = lax.dot_general(qf_ref[...], qblk,
                          (((1,), (1,)), ((), ())))
    shifted = pltpu.roll(neg, 1, 1)
    lane = lax.broadcasted_iota(jnp.int32, (_ROWS, _CB), 1)
    out_l_ref[...] = jnp.where(lane == 0, carry_ref[...], shifted) * _INV_T
    carry_ref[...] = neg[:, _CB - 1:_CB]

    @pl.when(i < _NQ)
    def _():
        start = jnp.clip(ptr_ref[0], 0, _Q - _ROWS)
        lo = i * _CB
        row = lax.broadcasted_iota(jnp.int32, (_CB, 1), 0) + lo
        sel = row - start
        overlap = jnp.logical_and(start < lo + _CB, start + _ROWS > lo)

        @pl.when(overlap)
        def _():
            j = lax.broadcasted_iota(jnp.int32, (_CB, _ROWS), 1)
            onehot = (sel == j).astype(jnp.float32)
            repl = lax.dot(onehot, kf_ref[...])
            mask = jnp.logical_and(sel >= 0, sel < _ROWS)
            out_q_ref[...] = jnp.where(mask, repl, qblk)

        @pl.when(jnp.logical_not(overlap))
        def _():
            out_q_ref[...] = qblk


def _moco_call(ptr, xq, xk, W_q, W_k, queue, interpret=False):
    return pl.pallas_call(
        _moco_body,
        grid=(_GRID,),
        in_specs=[
            pl.BlockSpec(memory_space=pltpu.SMEM),
            pl.BlockSpec((_ROWS, _DIN), lambda i: (0, 0)),
            pl.BlockSpec((_ROWS, _DIN), lambda i: (0, 0)),
            pl.BlockSpec((_DIN, _F), lambda i: (0, 0)),
            pl.BlockSpec((_DIN, _F), lambda i: (0, 0)),
            pl.BlockSpec((_CB, _F), lambda i: (jnp.minimum(i, _NQ - 1), 0)),
        ],
        out_specs=[
            pl.BlockSpec((_ROWS, _CB), lambda i: (0, i)),
            pl.BlockSpec((_CB, _F), lambda i: (jnp.minimum(i, _NQ - 1), 0)),
        ],
        out_shape=[
            jax.ShapeDtypeStruct((_ROWS, _Q + 1), jnp.float32),
            jax.ShapeDtypeStruct((_Q, _F), jnp.float32),
        ],
        scratch_shapes=[
            pltpu.VMEM((_ROWS, _F), jnp.float32),
            pltpu.VMEM((_ROWS, _F), jnp.float32),
            pltpu.VMEM((_ROWS, 1), jnp.float32),
        ],
        compiler_params=pltpu.CompilerParams(
            dimension_semantics=("arbitrary",)),
        interpret=interpret,
    )(ptr, xq, xk, W_q, W_k, queue)


def kernel(query_inputs, key_inputs, query_offset_x, query_offset_y,
           key_offset_x, key_offset_y, key_flipped, key_rotations,
           W_q, W_k, queue, queue_pointer):
    offs_q = (query_offset_x + query_offset_y)[:, None, None]
    xq = (query_inputs + offs_q).reshape(_ROWS, _DIN)
    flip = jnp.where(key_flipped, -1.0, 1.0)[:, None, None]
    offs_k = (key_offset_x + key_offset_y)[:, None, None]
    xk = (key_inputs * flip + offs_k).reshape(_ROWS, _DIN)
    ptr = jnp.asarray(queue_pointer, jnp.int32).reshape(1)

    logits, new_queue = _moco_call(ptr, xq, xk, W_q, W_k, queue)
    new_pointer = jnp.int32((queue_pointer + _ROWS) % _Q)
    return logits, new_queue, new_pointer
